# baseline (speedup 1.0000x reference)
import jax
import jax.numpy as jnp
from jax.experimental import pallas as pl
from jax.experimental.pallas import tpu as pltpu

_NUM_EMBEDDINGS = 1024
_DIM = 64
_COMMITMENT = 0.25
_ROWS = 2
_TILE = _ROWS * 1024


def _vq_half(xh, et, est, e2col):
    x2t = jnp.sum(xh * xh, axis=0, keepdims=True)
    m2t = jax.lax.dot_general(est, xh, (((1,), (0,)), ((), ())),
                              preferred_element_type=jnp.float32)
    d = (x2t + e2col) + m2t

    dmin = jnp.min(d, axis=0, keepdims=True)
    ids = jax.lax.broadcasted_iota(jnp.int32, d.shape, 0)
    idx = jnp.min(jnp.where(d == dmin, ids, _NUM_EMBEDDINGS),
                  axis=0, keepdims=True)

    onehot_t = (ids == idx).astype(jnp.float32)
    qt = jax.lax.dot_general(et, onehot_t, (((1,), (0,)), ((), ())),
                             preferred_element_type=jnp.float32)

    diff = qt - xh
    part = jnp.sum(diff * diff)
    return idx, xh + (qt - xh), part


def _vq_tile_kernel(xt_ref, et_ref, e_ref, qt_ref, idx_ref, loss_ref,
                    est_ref, e2_ref):
    et = et_ref[...]

    @pl.when(pl.program_id(0) == 0)
    def _prep():
        e = e_ref[...]
        est_ref[...] = e * -2.0
        e2_ref[...] = jnp.sum(e * e, axis=1, keepdims=True)

    est = est_ref[...]
    e2col = e2_ref[...]

    total = jnp.float32(0.0)
    for r in range(_ROWS):
        idx, out, part = _vq_half(xt_ref[r], et, est, e2col)
        qt_ref[r] = out
        idx_ref[pl.ds(r * 1024, 1024)] = idx.reshape(1024)
        total = total + part

    @pl.when(pl.program_id(0) == 0)
    def _init():
        loss_ref[0, 0] = 0.0

    loss_ref[0, 0] += total


def kernel(x, embedding):
    tokens = x.shape[0] * x.shape[1]
    grid = x.shape[0] // _ROWS
    xt = jnp.transpose(x, (0, 2, 1))
    et = embedding.T

    qt, idx, loss_sum = pl.pallas_call(
        _vq_tile_kernel,
        grid=(grid,),
        in_specs=[
            pl.BlockSpec((_ROWS, _DIM, 1024), lambda i: (i, 0, 0)),
            pl.BlockSpec((_DIM, _NUM_EMBEDDINGS), lambda i: (0, 0)),
            pl.BlockSpec((_NUM_EMBEDDINGS, _DIM), lambda i: (0, 0)),
        ],
        out_specs=[
            pl.BlockSpec((_ROWS, _DIM, 1024), lambda i: (i, 0, 0)),
            pl.BlockSpec((_TILE,), lambda i: (i,)),
            pl.BlockSpec(memory_space=pltpu.SMEM, block_shape=(1, 1),
                         index_map=lambda i: (0, 0)),
        ],
        out_shape=[
            jax.ShapeDtypeStruct((x.shape[0], _DIM, 1024), jnp.float32),
            jax.ShapeDtypeStruct((tokens,), jnp.int32),
            jax.ShapeDtypeStruct((1, 1), jnp.float32),
        ],
        scratch_shapes=[
            pltpu.VMEM((_NUM_EMBEDDINGS, _DIM), jnp.float32),
            pltpu.VMEM((_NUM_EMBEDDINGS, 1), jnp.float32),
        ],
    )(xt, et, embedding)

    q = jnp.transpose(qt, (0, 2, 1))
    mean_sq = loss_sum[0, 0] / (tokens * _DIM)
    loss = mean_sq + _COMMITMENT * mean_sq
    return (q, loss, idx)


# speedup vs baseline: 1.0190x; 1.0190x over previous
import jax
import jax.numpy as jnp
from jax.experimental import pallas as pl
from jax.experimental.pallas import tpu as pltpu

_NUM_EMBEDDINGS = 1024
_DIM = 64
_COMMITMENT = 0.25
_ROWS = 4
_TILE = _ROWS * 1024


def _vq_half(xh, et, est, e2col):
    x2t = jnp.sum(xh * xh, axis=0, keepdims=True)
    m2t = jax.lax.dot_general(est, xh, (((1,), (0,)), ((), ())),
                              preferred_element_type=jnp.float32)
    d = (x2t + e2col) + m2t

    dmin = jnp.min(d, axis=0, keepdims=True)
    ids = jax.lax.broadcasted_iota(jnp.int32, d.shape, 0)
    idx = jnp.min(jnp.where(d == dmin, ids, _NUM_EMBEDDINGS),
                  axis=0, keepdims=True)

    onehot_t = (ids == idx).astype(jnp.float32)
    qt = jax.lax.dot_general(et, onehot_t, (((1,), (0,)), ((), ())),
                             preferred_element_type=jnp.float32)

    diff = qt - xh
    part = jnp.sum(diff * diff)
    return idx, xh + (qt - xh), part


def _vq_tile_kernel(xt_ref, et_ref, e_ref, qt_ref, idx_ref, loss_ref,
                    est_ref, e2_ref):
    et = et_ref[...]

    @pl.when(pl.program_id(0) == 0)
    def _prep():
        e = e_ref[...]
        est_ref[...] = e * -2.0
        e2_ref[...] = jnp.sum(e * e, axis=1, keepdims=True)

    est = est_ref[...]
    e2col = e2_ref[...]

    total = jnp.float32(0.0)
    for r in range(_ROWS):
        idx, out, part = _vq_half(xt_ref[r], et, est, e2col)
        qt_ref[r] = out
        idx_ref[pl.ds(r * 1024, 1024)] = idx.reshape(1024)
        total = total + part

    @pl.when(pl.program_id(0) == 0)
    def _init():
        loss_ref[0, 0] = 0.0

    loss_ref[0, 0] += total


def kernel(x, embedding):
    tokens = x.shape[0] * x.shape[1]
    grid = x.shape[0] // _ROWS
    xt = jnp.transpose(x, (0, 2, 1))
    et = embedding.T

    qt, idx, loss_sum = pl.pallas_call(
        _vq_tile_kernel,
        grid=(grid,),
        in_specs=[
            pl.BlockSpec((_ROWS, _DIM, 1024), lambda i: (i, 0, 0)),
            pl.BlockSpec((_DIM, _NUM_EMBEDDINGS), lambda i: (0, 0)),
            pl.BlockSpec((_NUM_EMBEDDINGS, _DIM), lambda i: (0, 0)),
        ],
        out_specs=[
            pl.BlockSpec((_ROWS, _DIM, 1024), lambda i: (i, 0, 0)),
            pl.BlockSpec((_TILE,), lambda i: (i,)),
            pl.BlockSpec(memory_space=pltpu.SMEM, block_shape=(1, 1),
                         index_map=lambda i: (0, 0)),
        ],
        out_shape=[
            jax.ShapeDtypeStruct((x.shape[0], _DIM, 1024), jnp.float32),
            jax.ShapeDtypeStruct((tokens,), jnp.int32),
            jax.ShapeDtypeStruct((1, 1), jnp.float32),
        ],
        scratch_shapes=[
            pltpu.VMEM((_NUM_EMBEDDINGS, _DIM), jnp.float32),
            pltpu.VMEM((_NUM_EMBEDDINGS, 1), jnp.float32),
        ],
    )(xt, et, embedding)

    q = jnp.transpose(qt, (0, 2, 1))
    mean_sq = loss_sum[0, 0] / (tokens * _DIM)
    loss = mean_sq + _COMMITMENT * mean_sq
    return (q, loss, idx)


# speedup vs baseline: 1.0515x; 1.0319x over previous
import jax
import jax.numpy as jnp
from jax.experimental import pallas as pl
from jax.experimental.pallas import tpu as pltpu

_NUM_EMBEDDINGS = 1024
_DIM = 64
_COMMITMENT = 0.25
_ROWS = 4
_TILE = _ROWS * 1024


def _vq_half(xh, et, est, e2col):
    x2t = jnp.sum(xh * xh, axis=0, keepdims=True)
    m2t = jax.lax.dot_general(est, xh, (((1,), (0,)), ((), ())),
                              preferred_element_type=jnp.float32)
    d = (x2t + e2col) + m2t

    dmin = jnp.min(d, axis=0, keepdims=True)
    ids = jax.lax.broadcasted_iota(jnp.int32, d.shape, 0)
    idx = jnp.min(jnp.where(d == dmin, ids, _NUM_EMBEDDINGS),
                  axis=0, keepdims=True)

    onehot_t = (ids == idx).astype(jnp.float32)
    qt = jax.lax.dot_general(et, onehot_t, (((1,), (0,)), ((), ())),
                             preferred_element_type=jnp.float32)

    diff = qt - xh
    part = jnp.sum(diff * diff)
    return idx, xh + (qt - xh), part


def _vq_tile_kernel(xt_ref, et_ref, qt_ref, idx_ref, loss_ref,
                    est_ref, e2_ref):
    et = et_ref[...]

    @pl.when(pl.program_id(0) == 0)
    def _prep():
        e = jnp.transpose(et)
        est_ref[...] = e * -2.0
        e2_ref[...] = jnp.sum(e * e, axis=1, keepdims=True)

    est = est_ref[...]
    e2col = e2_ref[...]

    total = jnp.float32(0.0)
    for r in range(_ROWS):
        idx, out, part = _vq_half(xt_ref[r], et, est, e2col)
        qt_ref[r] = out
        idx_ref[pl.ds(r * 1024, 1024)] = idx.reshape(1024)
        total = total + part

    @pl.when(pl.program_id(0) == 0)
    def _init():
        loss_ref[0, 0] = 0.0

    loss_ref[0, 0] += total


def kernel(x, embedding):
    tokens = x.shape[0] * x.shape[1]
    grid = x.shape[0] // _ROWS
    xt = jnp.transpose(x, (0, 2, 1))
    et = embedding.T

    qt, idx, loss_sum = pl.pallas_call(
        _vq_tile_kernel,
        grid=(grid,),
        in_specs=[
            pl.BlockSpec((_ROWS, _DIM, 1024), lambda i: (i, 0, 0)),
            pl.BlockSpec((_DIM, _NUM_EMBEDDINGS), lambda i: (0, 0)),
        ],
        out_specs=[
            pl.BlockSpec((_ROWS, _DIM, 1024), lambda i: (i, 0, 0)),
            pl.BlockSpec((_TILE,), lambda i: (i,)),
            pl.BlockSpec(memory_space=pltpu.SMEM, block_shape=(1, 1),
                         index_map=lambda i: (0, 0)),
        ],
        out_shape=[
            jax.ShapeDtypeStruct((x.shape[0], _DIM, 1024), jnp.float32),
            jax.ShapeDtypeStruct((tokens,), jnp.int32),
            jax.ShapeDtypeStruct((1, 1), jnp.float32),
        ],
        scratch_shapes=[
            pltpu.VMEM((_NUM_EMBEDDINGS, _DIM), jnp.float32),
            pltpu.VMEM((_NUM_EMBEDDINGS, 1), jnp.float32),
        ],
    )(xt, et)

    q = jnp.transpose(qt, (0, 2, 1))
    mean_sq = loss_sum[0, 0] / (tokens * _DIM)
    loss = mean_sq + _COMMITMENT * mean_sq
    return (q, loss, idx)


# speedup vs baseline: 1.0586x; 1.0068x over previous
import jax
import jax.numpy as jnp
from jax.experimental import pallas as pl
from jax.experimental.pallas import tpu as pltpu

_NUM_EMBEDDINGS = 1024
_DIM = 64
_COMMITMENT = 0.25
_ROWS = 8
_TILE = _ROWS * 1024


def _vq_half(xh, et, est, e2col):
    x2t = jnp.sum(xh * xh, axis=0, keepdims=True)
    m2t = jax.lax.dot_general(est, xh, (((1,), (0,)), ((), ())),
                              preferred_element_type=jnp.float32)
    d = (x2t + e2col) + m2t

    dmin = jnp.min(d, axis=0, keepdims=True)
    ids = jax.lax.broadcasted_iota(jnp.int32, d.shape, 0)
    idx = jnp.min(jnp.where(d == dmin, ids, _NUM_EMBEDDINGS),
                  axis=0, keepdims=True)

    onehot_t = (ids == idx).astype(jnp.float32)
    qt = jax.lax.dot_general(et, onehot_t, (((1,), (0,)), ((), ())),
                             preferred_element_type=jnp.float32)

    diff = qt - xh
    part = jnp.sum(diff * diff)
    return idx, xh + (qt - xh), part


def _vq_tile_kernel(xt_ref, et_ref, qt_ref, idx_ref, loss_ref,
                    est_ref, e2_ref):
    et = et_ref[...]

    @pl.when(pl.program_id(0) == 0)
    def _prep():
        e = jnp.transpose(et)
        est_ref[...] = e * -2.0
        e2_ref[...] = jnp.sum(e * e, axis=1, keepdims=True)

    est = est_ref[...]
    e2col = e2_ref[...]

    total = jnp.float32(0.0)
    for r in range(_ROWS):
        idx, out, part = _vq_half(xt_ref[r], et, est, e2col)
        qt_ref[r] = out
        idx_ref[pl.ds(r * 1024, 1024)] = idx.reshape(1024)
        total = total + part

    @pl.when(pl.program_id(0) == 0)
    def _init():
        loss_ref[0, 0] = 0.0

    loss_ref[0, 0] += total


def kernel(x, embedding):
    tokens = x.shape[0] * x.shape[1]
    grid = x.shape[0] // _ROWS
    xt = jnp.transpose(x, (0, 2, 1))
    et = embedding.T

    qt, idx, loss_sum = pl.pallas_call(
        _vq_tile_kernel,
        grid=(grid,),
        in_specs=[
            pl.BlockSpec((_ROWS, _DIM, 1024), lambda i: (i, 0, 0)),
            pl.BlockSpec((_DIM, _NUM_EMBEDDINGS), lambda i: (0, 0)),
        ],
        out_specs=[
            pl.BlockSpec((_ROWS, _DIM, 1024), lambda i: (i, 0, 0)),
            pl.BlockSpec((_TILE,), lambda i: (i,)),
            pl.BlockSpec(memory_space=pltpu.SMEM, block_shape=(1, 1),
                         index_map=lambda i: (0, 0)),
        ],
        out_shape=[
            jax.ShapeDtypeStruct((x.shape[0], _DIM, 1024), jnp.float32),
            jax.ShapeDtypeStruct((tokens,), jnp.int32),
            jax.ShapeDtypeStruct((1, 1), jnp.float32),
        ],
        scratch_shapes=[
            pltpu.VMEM((_NUM_EMBEDDINGS, _DIM), jnp.float32),
            pltpu.VMEM((_NUM_EMBEDDINGS, 1), jnp.float32),
        ],
    )(xt, et)

    q = jnp.transpose(qt, (0, 2, 1))
    mean_sq = loss_sum[0, 0] / (tokens * _DIM)
    loss = mean_sq + _COMMITMENT * mean_sq
    return (q, loss, idx)
